# baseline (speedup 1.0000x reference)
import functools

import jax
import jax.numpy as jnp
from jax import lax
from jax.experimental import pallas as pl
from jax.experimental.pallas import tpu as pltpu
from jax.experimental.pallas import tpu_sc as plsc

LANES = 16
NBUF = 4


def kernel(inputs, token_table, pos_table):
    B, L = inputs.shape
    V, D = token_table.shape
    idx_t = inputs.T

    info = plsc.get_sparse_core_info()
    NC, NS = info.num_cores, info.num_subcores
    NW = NC * NS
    SW = B // NW
    mesh = plsc.VectorSubcoreMesh(core_axis_name="c", subcore_axis_name="s")

    @functools.partial(
        pl.kernel,
        mesh=mesh,
        compiler_params=pltpu.CompilerParams(
            use_tc_tiling_on_sc=False, needs_layout_passes=False),
        out_type=jax.ShapeDtypeStruct((L, D, B), jnp.float32),
        scratch_types=[
            pltpu.VMEM((L, SW), jnp.int32),
            pltpu.VMEM((L, D), jnp.float32),
        ] + [pltpu.VMEM((SW, D), jnp.float32) for _ in range(NBUF)]
          + [pltpu.VMEM((D, SW), jnp.float32) for _ in range(NBUF)]
          + [pltpu.SemaphoreType.DMA for _ in range(2 * NBUF)],
    )
    def k(idx_hbm, tab_hbm, pos_hbm, out_hbm, idx_v, pos_v, *bufs):
        rbufs = bufs[0:NBUF]
        tbufs = bufs[NBUF:2 * NBUF]
        gsems = bufs[2 * NBUF:3 * NBUF]
        osems = bufs[3 * NBUF:4 * NBUF]
        wid = lax.axis_index("s") * NC + lax.axis_index("c")
        col0 = wid * SW
        pltpu.sync_copy(idx_hbm.at[:, pl.ds(col0, SW)], idx_v)
        pltpu.sync_copy(pos_hbm, pos_v)

        def gather_start(l, b):
            pltpu.async_copy(tab_hbm.at[idx_v.at[l]], rbufs[b], gsems[b])

        def gather_wait(l, b):
            pltpu.make_async_copy(
                tab_hbm.at[idx_v.at[l]], rbufs[b], gsems[b]).wait()

        def out_start(l, b):
            pltpu.async_copy(
                tbufs[b], out_hbm.at[l, :, pl.ds(col0, SW)], osems[b])

        def out_wait(l, b):
            pltpu.make_async_copy(
                tbufs[b], out_hbm.at[l, :, pl.ds(col0, SW)], osems[b]).wait()

        for b in range(NBUF):
            gather_start(b, b)

        n_iter = L // NBUF
        lane_iota = lax.iota(jnp.int32, LANES)

        def iter_body(i, carry):
            for b in range(NBUF):
                l = i * NBUF + b
                gather_wait(l, b)

                @pl.when(i >= 1)
                def _():
                    out_wait(l - NBUF, b)

                rbuf, tbuf = rbufs[b], tbufs[b]
                pvs = [pos_v[l, pl.ds(j * LANES, LANES)] for j in range(D // LANES)]

                def r_body(r, c):
                    ridx = jnp.bitwise_and(lane_iota + r, SW - 1)
                    for j in range(D // LANES):
                        didx = lane_iota + (j * LANES)
                        vals = plsc.load_gather(rbuf, [ridx, didx]) + pvs[j]
                        plsc.store_scatter(tbuf, [didx, ridx], vals)
                    return c

                lax.fori_loop(0, SW, r_body, 0, unroll=2)
                out_start(l, b)

                @pl.when(i < n_iter - 1)
                def _():
                    gather_start(l + NBUF, b)

            return carry

        lax.fori_loop(0, n_iter, iter_body, 0)
        for b in range(NBUF):
            out_wait(L - NBUF + b, b)

    out_q = k(idx_t, token_table, pos_table)
    return jnp.transpose(out_q, (2, 0, 1))


# speedup vs baseline: 1.8906x; 1.8906x over previous
import functools

import jax
import jax.numpy as jnp
from jax import lax
from jax.experimental import pallas as pl
from jax.experimental.pallas import tpu as pltpu
from jax.experimental.pallas import tpu_sc as plsc

LANES = 16
NBUF = 4


def kernel(inputs, token_table, pos_table):
    B, L = inputs.shape
    V, D = token_table.shape
    idx_t = inputs.T

    info = plsc.get_sparse_core_info()
    NC, NS = info.num_cores, info.num_subcores
    NW = NC * NS
    SW = B // NW
    mesh = plsc.VectorSubcoreMesh(core_axis_name="c", subcore_axis_name="s")

    @functools.partial(
        pl.kernel,
        mesh=mesh,
        compiler_params=pltpu.CompilerParams(
            use_tc_tiling_on_sc=False, needs_layout_passes=False),
        out_type=jax.ShapeDtypeStruct((L, D, B), jnp.float32),
        scratch_types=[
            pltpu.VMEM((L, SW), jnp.int32),
            pltpu.VMEM((L, D), jnp.float32),
        ] + [pltpu.VMEM((SW, D), jnp.float32) for _ in range(NBUF)]
          + [pltpu.VMEM((D, SW), jnp.float32) for _ in range(NBUF)]
          + [pltpu.SemaphoreType.DMA for _ in range(2 * NBUF)],
    )
    def k(idx_hbm, tab_hbm, pos_hbm, out_hbm, idx_v, pos_v, *bufs):
        rbufs = bufs[0:NBUF]
        tbufs = bufs[NBUF:2 * NBUF]
        gsems = bufs[2 * NBUF:3 * NBUF]
        osems = bufs[3 * NBUF:4 * NBUF]
        wid = lax.axis_index("s") * NC + lax.axis_index("c")
        col0 = wid * SW
        pltpu.sync_copy(idx_hbm.at[:, pl.ds(col0, SW)], idx_v)
        pltpu.sync_copy(pos_hbm, pos_v)

        def gather_start(l, b):
            pltpu.async_copy(tab_hbm.at[idx_v.at[l]], rbufs[b], gsems[b])

        def gather_wait(l, b):
            pltpu.make_async_copy(
                tab_hbm.at[idx_v.at[l]], rbufs[b], gsems[b]).wait()

        def out_start(l, b):
            pltpu.async_copy(
                tbufs[b], out_hbm.at[l, :, pl.ds(col0, SW)], osems[b])

        def out_wait(l, b):
            pltpu.make_async_copy(
                tbufs[b], out_hbm.at[l, :, pl.ds(col0, SW)], osems[b]).wait()

        for b in range(NBUF):
            gather_start(b, b)

        n_iter = L // NBUF
        lane_iota = lax.iota(jnp.int32, LANES)

        def iter_body(i, carry):
            for b in range(NBUF):
                l = i * NBUF + b
                gather_wait(l, b)

                @pl.when(i >= 1)
                def _():
                    out_wait(l - NBUF, b)

                rbuf, tbuf = rbufs[b], tbufs[b]
                pvs = [pos_v[l, pl.ds(j * LANES, LANES)] for j in range(D // LANES)]

                @plsc.parallel_loop(0, SW, unroll=4)
                def r_body(r):
                    ridx = jnp.bitwise_and(lane_iota + r, SW - 1)
                    for j in range(D // LANES):
                        didx = lane_iota + (j * LANES)
                        vals = plsc.load_gather(rbuf, [ridx, didx]) + pvs[j]
                        plsc.store_scatter(tbuf, [didx, ridx], vals)
                out_start(l, b)

                @pl.when(i < n_iter - 1)
                def _():
                    gather_start(l + NBUF, b)

            return carry

        lax.fori_loop(0, n_iter, iter_body, 0)
        for b in range(NBUF):
            out_wait(L - NBUF + b, b)

    out_q = k(idx_t, token_table, pos_table)
    return jnp.transpose(out_q, (2, 0, 1))


# speedup vs baseline: 3.6844x; 1.9488x over previous
import functools

import jax
import jax.numpy as jnp
from jax import lax
from jax.experimental import pallas as pl
from jax.experimental.pallas import tpu as pltpu
from jax.experimental.pallas import tpu_sc as plsc

LANES = 16
NBUF = 4


def kernel(inputs, token_table, pos_table):
    B, L = inputs.shape
    V, D = token_table.shape
    idx_t = inputs.T

    info = plsc.get_sparse_core_info()
    NC, NS = info.num_cores, info.num_subcores
    NW = NC * NS
    SW = B // NW
    mesh = plsc.VectorSubcoreMesh(core_axis_name="c", subcore_axis_name="s")

    @functools.partial(
        pl.kernel,
        mesh=mesh,
        compiler_params=pltpu.CompilerParams(
            use_tc_tiling_on_sc=False, needs_layout_passes=False),
        out_type=jax.ShapeDtypeStruct((L, D // 8, B // SW, 8 * SW), jnp.float32),
        scratch_types=[
            pltpu.VMEM((L, SW), jnp.int32),
            pltpu.VMEM((L, D), jnp.float32),
        ] + [pltpu.VMEM((SW, D), jnp.float32) for _ in range(NBUF)]
          + [pltpu.VMEM((D // 8, 8 * SW), jnp.float32) for _ in range(NBUF)]
          + [pltpu.SemaphoreType.DMA for _ in range(2 * NBUF)],
    )
    def k(idx_hbm, tab_hbm, pos_hbm, out_hbm, idx_v, pos_v, *bufs):
        rbufs = bufs[0:NBUF]
        tbufs = bufs[NBUF:2 * NBUF]
        gsems = bufs[2 * NBUF:3 * NBUF]
        osems = bufs[3 * NBUF:4 * NBUF]
        wid = lax.axis_index("s") * NC + lax.axis_index("c")
        col0 = wid * SW
        pltpu.sync_copy(idx_hbm.at[:, pl.ds(col0, SW)], idx_v)
        pltpu.sync_copy(pos_hbm, pos_v)

        def gather_start(l, b):
            pltpu.async_copy(tab_hbm.at[idx_v.at[l]], rbufs[b], gsems[b])

        def gather_wait(l, b):
            pltpu.make_async_copy(
                tab_hbm.at[idx_v.at[l]], rbufs[b], gsems[b]).wait()

        def out_start(l, b):
            pltpu.async_copy(tbufs[b], out_hbm.at[l, :, wid], osems[b])

        def out_wait(l, b):
            pltpu.make_async_copy(
                tbufs[b], out_hbm.at[l, :, wid], osems[b]).wait()

        for b in range(NBUF):
            gather_start(b, b)

        n_iter = L // NBUF
        lane_iota = lax.iota(jnp.int32, LANES)
        cbase = jnp.bitwise_and(lane_iota, 7) * SW
        dtbase = lax.shift_right_logical(lane_iota, 3)

        def iter_body(i, carry):
            for b in range(NBUF):
                l = i * NBUF + b
                gather_wait(l, b)

                @pl.when(i >= 1)
                def _():
                    out_wait(l - NBUF, b)

                rbuf, tbuf = rbufs[b], tbufs[b]
                pvs = [pos_v[l, pl.ds(j * LANES, LANES)] for j in range(D // LANES)]

                @plsc.parallel_loop(0, SW, unroll=4)
                def r_body(r):
                    ridx = jnp.bitwise_and(lane_iota + r, SW - 1)
                    cidx = cbase + ridx
                    for j in range(D // LANES):
                        didx = lane_iota + (j * LANES)
                        vals = plsc.load_gather(rbuf, [ridx, didx]) + pvs[j]
                        plsc.store_scatter(tbuf, [dtbase + 2 * j, cidx], vals)
                out_start(l, b)

                @pl.when(i < n_iter - 1)
                def _():
                    gather_start(l + NBUF, b)

            return carry

        lax.fori_loop(0, n_iter, iter_body, 0)
        for b in range(NBUF):
            out_wait(L - NBUF + b, b)

    out4 = k(idx_t, token_table, pos_table)
    q5 = out4.reshape(L, D // 8, B // SW, 8, SW)
    q6 = jnp.transpose(q5, (0, 1, 3, 2, 4))
    q = q6.reshape(L, D, B)
    return jnp.transpose(q, (2, 0, 1))
